# baseline (speedup 1.0000x reference)
import functools

import jax
import jax.numpy as jnp
from jax import lax
from jax.experimental import pallas as pl
from jax.experimental.pallas import tpu as pltpu
from jax.experimental.pallas import tpu_sc as plsc

N_CORES = 2
N_SUBCORES = 16
N_WORKERS = N_CORES * N_SUBCORES
CHUNK = 128


def _sc_gather_body(nodes_hbm, s_hbm, r_hbm, inc_hbm, outg_hbm,
                    idx_v, rows0, rows1, g0, g1, w0, w1):
    c = lax.axis_index("c")
    s = lax.axis_index("s")
    wid = s * N_CORES + c
    e = s_hbm.shape[0]
    per_w = e // N_WORKERS
    base = wid * per_w
    n_full = per_w // CHUNK
    tail = per_w - n_full * CHUNK

    chunks = [(k * CHUNK, CHUNK) for k in range(n_full)]
    if tail:
        chunks.append((n_full * CHUNK, tail))
    bufs = (rows0, rows1)
    gsem = (g0, g1)
    wsem = (w0, w1)

    def run(idx_hbm, dst_hbm):
        pltpu.sync_copy(idx_hbm.at[pl.ds(base, per_w)], idx_v)
        t_n = len(chunks)

        def start_g(k):
            off, sz = chunks[k]
            b = bufs[k % 2]
            return pltpu.async_copy(
                nodes_hbm.at[idx_v.at[pl.ds(off, sz)]],
                b.at[pl.ds(0, sz), :], gsem[k % 2])

        def start_w(k):
            off, sz = chunks[k]
            b = bufs[k % 2]
            return pltpu.async_copy(
                b.at[pl.ds(0, sz), :],
                dst_hbm.at[pl.ds(base + off, sz), :], wsem[k % 2])

        hw = [None] * t_n
        hg = start_g(0)
        for k in range(t_n):
            hg.wait()
            if k + 1 < t_n:
                if k >= 1:
                    hw[k - 1].wait()
                hg = start_g(k + 1)
            hw[k] = start_w(k)
        if t_n >= 2:
            hw[t_n - 2].wait()
        hw[t_n - 1].wait()

    run(s_hbm, inc_hbm)
    run(r_hbm, outg_hbm)


def _sc_gather(nodes, senders, receivers):
    e = senders.shape[0]
    d = nodes.shape[1]
    per_w = e // N_WORKERS
    assert per_w * N_WORKERS == e and per_w % 8 == 0
    mesh = plsc.VectorSubcoreMesh(core_axis_name="c", subcore_axis_name="s")
    fn = functools.partial(
        pl.kernel,
        mesh=mesh,
        out_type=(
            jax.ShapeDtypeStruct((e, d), jnp.float32),
            jax.ShapeDtypeStruct((e, d), jnp.float32),
        ),
        scratch_types=[
            pltpu.VMEM((per_w,), jnp.int32),
            pltpu.VMEM((CHUNK, d), jnp.float32),
            pltpu.VMEM((CHUNK, d), jnp.float32),
            pltpu.SemaphoreType.DMA,
            pltpu.SemaphoreType.DMA,
            pltpu.SemaphoreType.DMA,
            pltpu.SemaphoreType.DMA,
        ],
    )(_sc_gather_body)
    return fn(nodes, senders, receivers)


def _sc_scatter_body(msg_hbm, r_hbm, aggp_hbm, zbuf, rows_v, idx_v,
                     rows_t, idx_t, agg_sh):
    c = lax.axis_index("c")
    s = lax.axis_index("s")
    n = agg_sh.shape[0]
    per_tile_n = n // N_SUBCORES
    row0 = s * per_tile_n

    def zrow(i, carry):
        for j in range(8):
            zbuf[i, pl.ds(j * 16, 16)] = jnp.zeros((16,), jnp.float32)
        return carry
    lax.fori_loop(0, CHUNK, zrow, 0)

    def zcp(i, carry):
        pltpu.sync_copy(zbuf, agg_sh.at[pl.ds(row0 + i * CHUNK, CHUNK), :])
        return carry
    lax.fori_loop(0, per_tile_n // CHUNK, zcp, 0)
    plsc.subcore_barrier()

    e = r_hbm.shape[0]
    per_core = e // N_CORES
    per_tile = per_core // N_SUBCORES
    base = c * per_core + s * per_tile
    n_full = per_tile // CHUNK
    tail = per_tile - n_full * CHUNK

    def body(k, carry):
        off = base + k * CHUNK
        pltpu.sync_copy(r_hbm.at[pl.ds(off, CHUNK)], idx_v)
        pltpu.sync_copy(msg_hbm.at[pl.ds(off, CHUNK), :], rows_v)
        pltpu.sync_copy(rows_v, agg_sh.at[idx_v], add=True)
        return carry
    lax.fori_loop(0, n_full, body, 0)
    if tail:
        off = base + n_full * CHUNK
        pltpu.sync_copy(r_hbm.at[pl.ds(off, tail)], idx_t)
        pltpu.sync_copy(msg_hbm.at[pl.ds(off, tail), :], rows_t)
        pltpu.sync_copy(rows_t, agg_sh.at[idx_t], add=True)
    plsc.subcore_barrier()

    def wcp(i, carry):
        pltpu.sync_copy(agg_sh.at[pl.ds(row0 + i * CHUNK, CHUNK), :], zbuf)
        pltpu.sync_copy(zbuf, aggp_hbm.at[pl.ds(c * n + row0 + i * CHUNK, CHUNK), :])
        return carry
    lax.fori_loop(0, per_tile_n // CHUNK, wcp, 0)


def _sc_scatter(msg, receivers, n_pad):
    e, d = msg.shape
    per_tile = e // (N_CORES * N_SUBCORES)
    assert per_tile * N_CORES * N_SUBCORES == e and per_tile % 8 == 0
    tail = per_tile % CHUNK
    mesh = plsc.VectorSubcoreMesh(core_axis_name="c", subcore_axis_name="s")
    fn = functools.partial(
        pl.kernel,
        mesh=mesh,
        out_type=jax.ShapeDtypeStruct((N_CORES * n_pad, d), jnp.float32),
        scratch_types=[
            pltpu.VMEM((CHUNK, d), jnp.float32),
            pltpu.VMEM((CHUNK, d), jnp.float32),
            pltpu.VMEM((CHUNK,), jnp.int32),
            pltpu.VMEM((max(tail, 8), d), jnp.float32),
            pltpu.VMEM((max(tail, 8),), jnp.int32),
            pltpu.VMEM_SHARED((n_pad, d), jnp.float32),
        ],
    )(_sc_scatter_body)
    return fn(msg, receivers)


def _contract_attr(attr, t, a_dim, d):
    acc = attr[:, 0:1] * t[:, 0:d]
    for a in range(1, a_dim):
        acc = acc + attr[:, a:a + 1] * t[:, a * d:(a + 1) * d]
    return acc


def _msg_body(inc_ref, outg_ref, add_ref, ea_ref,
              ws_ref, wr_ref, wa_ref, wgs_ref, wgr_ref, wga_ref,
              w1_ref, w1g_ref, o_ref):
    d = inc_ref.shape[1]
    a_dim = ea_ref.shape[1]
    inc = inc_ref[...]
    outg = outg_ref[...]
    add = add_ref[...]
    ea = ea_ref[...]
    t = (jnp.dot(inc, ws_ref[...], preferred_element_type=jnp.float32)
         + jnp.dot(outg, wr_ref[...], preferred_element_type=jnp.float32)
         + jnp.dot(add, wa_ref[...], preferred_element_type=jnp.float32))
    tg = (jnp.dot(inc, wgs_ref[...], preferred_element_type=jnp.float32)
          + jnp.dot(outg, wgr_ref[...], preferred_element_type=jnp.float32)
          + jnp.dot(add, wga_ref[...], preferred_element_type=jnp.float32))
    h = _contract_attr(ea, t, a_dim, d)
    hg = _contract_attr(ea, tg, a_dim, d)
    m0 = h * jax.nn.sigmoid(hg)
    t1 = jnp.dot(m0, w1_ref[...], preferred_element_type=jnp.float32)
    t1g = jnp.dot(m0, w1g_ref[...], preferred_element_type=jnp.float32)
    h1 = _contract_attr(ea, t1, a_dim, d)
    h1g = _contract_attr(ea, t1g, a_dim, d)
    o_ref[...] = h1 * jax.nn.sigmoid(h1g)


def _tc_message(inc, outg, add_p, ea_p, wm0, wm0g, wm1, wm1g):
    e_pad, d = inc.shape
    d_add = add_p.shape[1]
    a_dim = ea_p.shape[1]
    ak = a_dim * d
    be = 1280
    assert e_pad % be == 0
    grid = (e_pad // be,)
    row_spec = lambda w: pl.BlockSpec((be, w), lambda i: (i, 0))
    full = lambda r: pl.BlockSpec((r, ak), lambda i: (0, 0))
    return pl.pallas_call(
        _msg_body,
        grid=grid,
        in_specs=[
            row_spec(d), row_spec(d), row_spec(d_add), row_spec(a_dim),
            full(d), full(d), full(d_add),
            full(d), full(d), full(d_add),
            full(d), full(d),
        ],
        out_specs=pl.BlockSpec((be, d), lambda i: (i, 0)),
        out_shape=jax.ShapeDtypeStruct((e_pad, d), jnp.float32),
    )(inc, outg, add_p, ea_p,
      wm0[:d], wm0[d:2 * d], wm0[2 * d:],
      wm0g[:d], wm0g[d:2 * d], wm0g[2 * d:],
      wm1, wm1g)


def _upd_body(nodes_ref, aggp_ref, na_ref,
              w0n_ref, w0a_ref, w0gn_ref, w0ga_ref, w1_ref, o_ref):
    d = nodes_ref.shape[1]
    a_dim = na_ref.shape[1]
    nd = nodes_ref[...]
    agg = jnp.sum(aggp_ref[...], axis=0)
    na = na_ref[...]
    t = (jnp.dot(nd, w0n_ref[...], preferred_element_type=jnp.float32)
         + jnp.dot(agg, w0a_ref[...], preferred_element_type=jnp.float32))
    tg = (jnp.dot(nd, w0gn_ref[...], preferred_element_type=jnp.float32)
          + jnp.dot(agg, w0ga_ref[...], preferred_element_type=jnp.float32))
    x = _contract_attr(na, t, a_dim, d) * jax.nn.sigmoid(
        _contract_attr(na, tg, a_dim, d))
    t1 = jnp.dot(x, w1_ref[...], preferred_element_type=jnp.float32)
    o_ref[...] = nd + _contract_attr(na, t1, a_dim, d)


def _tc_update(nodes, aggp, nattr, wu0, wu0g, wu1):
    n, d = nodes.shape
    n_part = aggp.shape[0]
    a_dim = nattr.shape[1]
    ak = a_dim * d
    bn = 1000
    grid = (n // bn,)
    full = lambda r: pl.BlockSpec((r, ak), lambda i: (0, 0))
    return pl.pallas_call(
        _upd_body,
        grid=grid,
        in_specs=[
            pl.BlockSpec((bn, d), lambda i: (i, 0)),
            pl.BlockSpec((n_part, bn, d), lambda i: (0, i, 0)),
            pl.BlockSpec((bn, a_dim), lambda i: (i, 0)),
            full(d), full(d), full(d), full(d), full(d),
        ],
        out_specs=pl.BlockSpec((bn, d), lambda i: (i, 0)),
        out_shape=jax.ShapeDtypeStruct((n, d), jnp.float32),
    )(nodes, aggp, nattr,
      wu0[:d], wu0[d:], wu0g[:d], wu0g[d:], wu1)


def kernel(nodes, senders, receivers, node_attributes, edge_attributes,
           additional_message_features, Wm0, Wm0g, Wm1, Wm1g, Wu0, Wu0g, Wu1):
    n, d = nodes.shape
    e = senders.shape[0]
    a_dim = node_attributes.shape[1]
    d_add = additional_message_features.shape[1]

    wm0 = Wm0.reshape(2 * d + d_add, a_dim * d)
    wm0g = Wm0g.reshape(2 * d + d_add, a_dim * d)
    wm1 = Wm1.reshape(d, a_dim * d)
    wm1g = Wm1g.reshape(d, a_dim * d)
    wu0 = Wu0.reshape(2 * d, a_dim * d)
    wu0g = Wu0g.reshape(2 * d, a_dim * d)
    wu1 = Wu1.reshape(d, a_dim * d)

    n_gran = CHUNK * N_SUBCORES
    n_pad = ((n + n_gran - 1) // n_gran) * n_gran

    n_chunks = 5
    e_c = e // n_chunks
    assert e_c * n_chunks == e and e_c % (8 * N_WORKERS) == 0

    sls = [slice(k * e_c, (k + 1) * e_c) for k in range(n_chunks)]
    gathered = [_sc_gather(nodes, senders[sl], receivers[sl]) for sl in sls]
    msgs = [_tc_message(inc, outg, additional_message_features[sl],
                        edge_attributes[sl], wm0, wm0g, wm1, wm1g)
            for (inc, outg), sl in zip(gathered, sls)]
    parts = [_sc_scatter(msg, receivers[sl], n_pad).reshape(N_CORES, n_pad, d)
             for msg, sl in zip(msgs, sls)]
    aggp_all = jnp.concatenate(parts, axis=0)
    return _tc_update(nodes, aggp_all, node_attributes, wu0, wu0g, wu1)


# speedup vs baseline: 1.0216x; 1.0216x over previous
import functools

import jax
import jax.numpy as jnp
from jax import lax
from jax.experimental import pallas as pl
from jax.experimental.pallas import tpu as pltpu
from jax.experimental.pallas import tpu_sc as plsc

N_CORES = 2
N_SUBCORES = 16
N_WORKERS = N_CORES * N_SUBCORES
CHUNK = 128


def _sc_gather_body(nodes_hbm, s_hbm, r_hbm, inc_hbm, outg_hbm,
                    idx_v, rows0, g0):
    c = lax.axis_index("c")
    s = lax.axis_index("s")
    wid = s * N_CORES + c
    e = s_hbm.shape[0]
    per_w = e // N_WORKERS
    base = wid * per_w
    n_full = per_w // CHUNK
    tail = per_w - n_full * CHUNK

    def run(idx_hbm, dst_hbm):
        pltpu.sync_copy(idx_hbm.at[pl.ds(base, per_w)], idx_v)

        def body(k, carry):
            off = k * CHUNK
            pltpu.async_copy(nodes_hbm.at[idx_v.at[pl.ds(off, CHUNK)]],
                             rows0, g0).wait()
            pltpu.sync_copy(rows0, dst_hbm.at[pl.ds(base + off, CHUNK), :])
            return carry
        lax.fori_loop(0, n_full, body, 0)
        if tail:
            off = n_full * CHUNK
            pltpu.async_copy(nodes_hbm.at[idx_v.at[pl.ds(off, tail)]],
                             rows0.at[pl.ds(0, tail), :], g0).wait()
            pltpu.sync_copy(rows0.at[pl.ds(0, tail), :],
                            dst_hbm.at[pl.ds(base + off, tail), :])

    run(s_hbm, inc_hbm)
    run(r_hbm, outg_hbm)


def _sc_gather(nodes, senders, receivers):
    e = senders.shape[0]
    d = nodes.shape[1]
    per_w = e // N_WORKERS
    assert per_w * N_WORKERS == e and per_w % 8 == 0
    mesh = plsc.VectorSubcoreMesh(core_axis_name="c", subcore_axis_name="s")
    fn = functools.partial(
        pl.kernel,
        mesh=mesh,
        out_type=(
            jax.ShapeDtypeStruct((e, d), jnp.float32),
            jax.ShapeDtypeStruct((e, d), jnp.float32),
        ),
        scratch_types=[
            pltpu.VMEM((per_w,), jnp.int32),
            pltpu.VMEM((CHUNK, d), jnp.float32),
            pltpu.SemaphoreType.DMA,
        ],
    )(_sc_gather_body)
    return fn(nodes, senders, receivers)


def _sc_scatter_body(msg_hbm, r_hbm, aggp_hbm, zbuf, rows_v, idx_v,
                     rows_t, idx_t, agg_sh):
    c = lax.axis_index("c")
    s = lax.axis_index("s")
    n = agg_sh.shape[0]
    per_tile_n = n // N_SUBCORES
    row0 = s * per_tile_n

    def zrow(i, carry):
        for j in range(8):
            zbuf[i, pl.ds(j * 16, 16)] = jnp.zeros((16,), jnp.float32)
        return carry
    lax.fori_loop(0, CHUNK, zrow, 0)

    def zcp(i, carry):
        pltpu.sync_copy(zbuf, agg_sh.at[pl.ds(row0 + i * CHUNK, CHUNK), :])
        return carry
    lax.fori_loop(0, per_tile_n // CHUNK, zcp, 0)
    plsc.subcore_barrier()

    e = r_hbm.shape[0]
    per_core = e // N_CORES
    per_tile = per_core // N_SUBCORES
    base = c * per_core + s * per_tile
    n_full = per_tile // CHUNK
    tail = per_tile - n_full * CHUNK

    def body(k, carry):
        off = base + k * CHUNK
        pltpu.sync_copy(r_hbm.at[pl.ds(off, CHUNK)], idx_v)
        pltpu.sync_copy(msg_hbm.at[pl.ds(off, CHUNK), :], rows_v)
        pltpu.sync_copy(rows_v, agg_sh.at[idx_v], add=True)
        return carry
    lax.fori_loop(0, n_full, body, 0)
    if tail:
        off = base + n_full * CHUNK
        pltpu.sync_copy(r_hbm.at[pl.ds(off, tail)], idx_t)
        pltpu.sync_copy(msg_hbm.at[pl.ds(off, tail), :], rows_t)
        pltpu.sync_copy(rows_t, agg_sh.at[idx_t], add=True)
    plsc.subcore_barrier()

    def wcp(i, carry):
        pltpu.sync_copy(agg_sh.at[pl.ds(row0 + i * CHUNK, CHUNK), :], zbuf)
        pltpu.sync_copy(zbuf, aggp_hbm.at[pl.ds(c * n + row0 + i * CHUNK, CHUNK), :])
        return carry
    lax.fori_loop(0, per_tile_n // CHUNK, wcp, 0)


def _sc_scatter(msg, receivers, n_pad):
    e, d = msg.shape
    per_tile = e // (N_CORES * N_SUBCORES)
    assert per_tile * N_CORES * N_SUBCORES == e and per_tile % 8 == 0
    tail = per_tile % CHUNK
    mesh = plsc.VectorSubcoreMesh(core_axis_name="c", subcore_axis_name="s")
    fn = functools.partial(
        pl.kernel,
        mesh=mesh,
        out_type=jax.ShapeDtypeStruct((N_CORES * n_pad, d), jnp.float32),
        scratch_types=[
            pltpu.VMEM((CHUNK, d), jnp.float32),
            pltpu.VMEM((CHUNK, d), jnp.float32),
            pltpu.VMEM((CHUNK,), jnp.int32),
            pltpu.VMEM((max(tail, 8), d), jnp.float32),
            pltpu.VMEM((max(tail, 8),), jnp.int32),
            pltpu.VMEM_SHARED((n_pad, d), jnp.float32),
        ],
    )(_sc_scatter_body)
    return fn(msg, receivers)


def _contract_attr(attr, t, a_dim, d):
    acc = attr[:, 0:1] * t[:, 0:d]
    for a in range(1, a_dim):
        acc = acc + attr[:, a:a + 1] * t[:, a * d:(a + 1) * d]
    return acc


def _msg_body(inc_ref, outg_ref, add_ref, ea_ref,
              ws_ref, wr_ref, wa_ref, wgs_ref, wgr_ref, wga_ref,
              w1_ref, w1g_ref, o_ref):
    d = inc_ref.shape[1]
    a_dim = ea_ref.shape[1]
    inc = inc_ref[...]
    outg = outg_ref[...]
    add = add_ref[...]
    ea = ea_ref[...]
    t = (jnp.dot(inc, ws_ref[...], preferred_element_type=jnp.float32)
         + jnp.dot(outg, wr_ref[...], preferred_element_type=jnp.float32)
         + jnp.dot(add, wa_ref[...], preferred_element_type=jnp.float32))
    tg = (jnp.dot(inc, wgs_ref[...], preferred_element_type=jnp.float32)
          + jnp.dot(outg, wgr_ref[...], preferred_element_type=jnp.float32)
          + jnp.dot(add, wga_ref[...], preferred_element_type=jnp.float32))
    h = _contract_attr(ea, t, a_dim, d)
    hg = _contract_attr(ea, tg, a_dim, d)
    m0 = h * jax.nn.sigmoid(hg)
    t1 = jnp.dot(m0, w1_ref[...], preferred_element_type=jnp.float32)
    t1g = jnp.dot(m0, w1g_ref[...], preferred_element_type=jnp.float32)
    h1 = _contract_attr(ea, t1, a_dim, d)
    h1g = _contract_attr(ea, t1g, a_dim, d)
    o_ref[...] = h1 * jax.nn.sigmoid(h1g)


def _tc_message(inc, outg, add_p, ea_p, wm0, wm0g, wm1, wm1g):
    e_pad, d = inc.shape
    d_add = add_p.shape[1]
    a_dim = ea_p.shape[1]
    ak = a_dim * d
    be = 1280
    assert e_pad % be == 0
    grid = (e_pad // be,)
    row_spec = lambda w: pl.BlockSpec((be, w), lambda i: (i, 0))
    full = lambda r: pl.BlockSpec((r, ak), lambda i: (0, 0))
    return pl.pallas_call(
        _msg_body,
        grid=grid,
        in_specs=[
            row_spec(d), row_spec(d), row_spec(d_add), row_spec(a_dim),
            full(d), full(d), full(d_add),
            full(d), full(d), full(d_add),
            full(d), full(d),
        ],
        out_specs=pl.BlockSpec((be, d), lambda i: (i, 0)),
        out_shape=jax.ShapeDtypeStruct((e_pad, d), jnp.float32),
    )(inc, outg, add_p, ea_p,
      wm0[:d], wm0[d:2 * d], wm0[2 * d:],
      wm0g[:d], wm0g[d:2 * d], wm0g[2 * d:],
      wm1, wm1g)


def _upd_body(nodes_ref, aggp_ref, na_ref,
              w0n_ref, w0a_ref, w0gn_ref, w0ga_ref, w1_ref, o_ref):
    d = nodes_ref.shape[1]
    a_dim = na_ref.shape[1]
    nd = nodes_ref[...]
    agg = jnp.sum(aggp_ref[...], axis=0)
    na = na_ref[...]
    t = (jnp.dot(nd, w0n_ref[...], preferred_element_type=jnp.float32)
         + jnp.dot(agg, w0a_ref[...], preferred_element_type=jnp.float32))
    tg = (jnp.dot(nd, w0gn_ref[...], preferred_element_type=jnp.float32)
          + jnp.dot(agg, w0ga_ref[...], preferred_element_type=jnp.float32))
    x = _contract_attr(na, t, a_dim, d) * jax.nn.sigmoid(
        _contract_attr(na, tg, a_dim, d))
    t1 = jnp.dot(x, w1_ref[...], preferred_element_type=jnp.float32)
    o_ref[...] = nd + _contract_attr(na, t1, a_dim, d)


def _tc_update(nodes, aggp, nattr, wu0, wu0g, wu1):
    n, d = nodes.shape
    n_part = aggp.shape[0]
    a_dim = nattr.shape[1]
    ak = a_dim * d
    bn = 1000
    grid = (n // bn,)
    full = lambda r: pl.BlockSpec((r, ak), lambda i: (0, 0))
    return pl.pallas_call(
        _upd_body,
        grid=grid,
        in_specs=[
            pl.BlockSpec((bn, d), lambda i: (i, 0)),
            pl.BlockSpec((n_part, bn, d), lambda i: (0, i, 0)),
            pl.BlockSpec((bn, a_dim), lambda i: (i, 0)),
            full(d), full(d), full(d), full(d), full(d),
        ],
        out_specs=pl.BlockSpec((bn, d), lambda i: (i, 0)),
        out_shape=jax.ShapeDtypeStruct((n, d), jnp.float32),
    )(nodes, aggp, nattr,
      wu0[:d], wu0[d:], wu0g[:d], wu0g[d:], wu1)


def kernel(nodes, senders, receivers, node_attributes, edge_attributes,
           additional_message_features, Wm0, Wm0g, Wm1, Wm1g, Wu0, Wu0g, Wu1):
    n, d = nodes.shape
    e = senders.shape[0]
    a_dim = node_attributes.shape[1]
    d_add = additional_message_features.shape[1]

    wm0 = Wm0.reshape(2 * d + d_add, a_dim * d)
    wm0g = Wm0g.reshape(2 * d + d_add, a_dim * d)
    wm1 = Wm1.reshape(d, a_dim * d)
    wm1g = Wm1g.reshape(d, a_dim * d)
    wu0 = Wu0.reshape(2 * d, a_dim * d)
    wu0g = Wu0g.reshape(2 * d, a_dim * d)
    wu1 = Wu1.reshape(d, a_dim * d)

    n_gran = CHUNK * N_SUBCORES
    n_pad = ((n + n_gran - 1) // n_gran) * n_gran

    n_chunks = 5
    e_c = e // n_chunks
    assert e_c * n_chunks == e and e_c % (8 * N_WORKERS) == 0

    sls = [slice(k * e_c, (k + 1) * e_c) for k in range(n_chunks)]
    gathered = [_sc_gather(nodes, senders[sl], receivers[sl]) for sl in sls]
    msgs = [_tc_message(inc, outg, additional_message_features[sl],
                        edge_attributes[sl], wm0, wm0g, wm1, wm1g)
            for (inc, outg), sl in zip(gathered, sls)]
    parts = [_sc_scatter(msg, receivers[sl], n_pad).reshape(N_CORES, n_pad, d)
             for msg, sl in zip(msgs, sls)]
    aggp_all = jnp.concatenate(parts, axis=0)
    return _tc_update(nodes, aggp_all, node_attributes, wu0, wu0g, wu1)
